# baseline (speedup 1.0000x reference)
import functools
import jax
import jax.numpy as jnp
from jax import lax
from jax.experimental import pallas as pl
from jax.experimental.pallas import tpu as pltpu
from jax.experimental.pallas import tpu_sc as plsc

_D = 64
_SCALE = 8.0
_NC, _NS = 2, 16
_NW = _NC * _NS
_B = 4096 * 200
_BPW = _B // _NW
_CHUNK = 800
_NCHUNK = _BPW // _CHUNK

_mesh = plsc.VectorSubcoreMesh(
    core_axis_name="c", subcore_axis_name="s",
    num_cores=_NC, num_subcores=_NS)


def _gather_body(x_hbm, table_hbm, out_hbm, idx_v, rows_v, sem):
  wid = lax.axis_index("s") * _NC + lax.axis_index("c")
  base = wid * _BPW

  def chunk_body(g, carry):
    off = base + g * _CHUNK
    pltpu.sync_copy(x_hbm.at[pl.ds(off, _CHUNK)], idx_v)
    pltpu.async_copy(table_hbm.at[idx_v], rows_v, sem).wait()

    def scale_body(i, c):
      for j in range(_D // 16):
        sl = pl.ds(j * 16, 16)
        rows_v[i, sl] = rows_v[i, sl] * _SCALE
      return c

    lax.fori_loop(0, _CHUNK, scale_body, 0)
    pltpu.sync_copy(rows_v, out_hbm.at[pl.ds(off, _CHUNK)])
    return carry

  lax.fori_loop(0, _NCHUNK, chunk_body, 0)


_gather = pl.kernel(
    _gather_body,
    out_type=jax.ShapeDtypeStruct((_B, _D), jnp.float32),
    mesh=_mesh,
    scratch_types=[
        pltpu.VMEM((_CHUNK,), jnp.int32),
        pltpu.VMEM((_CHUNK, _D), jnp.float32),
        pltpu.SemaphoreType.DMA,
    ],
    compiler_params=pltpu.CompilerParams(use_tc_tiling_on_sc=False),
)


@jax.jit
def kernel(x, table):
  xf = x.reshape(-1).astype(jnp.int32)
  out = _gather(xf, table)
  return out.reshape(x.shape + (_D,))


# speedup vs baseline: 1.0856x; 1.0856x over previous
import jax
import jax.numpy as jnp
from jax import lax
from jax.experimental import pallas as pl
from jax.experimental.pallas import tpu as pltpu
from jax.experimental.pallas import tpu_sc as plsc

_D = 64
_SCALE = 8.0
_NC, _NS = 2, 16
_NW = _NC * _NS
_B = 4096 * 200
_BPW = _B // _NW
_CHUNK = 800
_NCHUNK = _BPW // _CHUNK

_mesh = plsc.VectorSubcoreMesh(
    core_axis_name="c", subcore_axis_name="s",
    num_cores=_NC, num_subcores=_NS)


def _gather_body(x_hbm, table_hbm, out_hbm, idx_v, rows0, rows1, gs0, gs1,
                 ss0, ss1):
  rows = (rows0, rows1)
  gsem = (gs0, gs1)
  ssem = (ss0, ss1)
  wid = lax.axis_index("s") * _NC + lax.axis_index("c")
  base = wid * _BPW

  pltpu.sync_copy(x_hbm.at[pl.ds(base, _BPW)], idx_v)

  def idx_slice(g):
    return idx_v.at[pl.ds(pl.multiple_of(g * _CHUNK, _CHUNK), _CHUNK)]

  def out_slice(g):
    return out_hbm.at[pl.ds(pl.multiple_of(base + g * _CHUNK, _CHUNK), _CHUNK)]

  def start_gather(g, k):
    return pltpu.async_copy(table_hbm.at[idx_slice(g)], rows[k], gsem[k])

  def wait_gather(g, k):
    pltpu.make_async_copy(table_hbm.at[idx_slice(g)], rows[k], gsem[k]).wait()

  def start_store(g, k):
    return pltpu.async_copy(rows[k], out_slice(g), ssem[k])

  def wait_store(g, k):
    pltpu.make_async_copy(rows[k], out_slice(g), ssem[k]).wait()

  def scale(k):
    buf = rows[k]

    def scale_row(i, c):
      for j in range(_D // 16):
        sl = pl.ds(j * 16, 16)
        buf[i, sl] = buf[i, sl] * _SCALE
      return c

    lax.fori_loop(0, _CHUNK, scale_row, 0)

  start_gather(0, 0)
  start_gather(1, 1)
  wait_gather(0, 0)
  scale(0)
  start_store(0, 0)

  def main_body(t, c):
    g = 2 * t + 1
    wait_store(g - 1, 0)
    start_gather(g + 1, 0)
    wait_gather(g, 1)
    scale(1)
    start_store(g, 1)

    g = 2 * t + 2
    wait_store(g - 1, 1)
    start_gather(g + 1, 1)
    wait_gather(g, 0)
    scale(0)
    start_store(g, 0)
    return c

  lax.fori_loop(0, (_NCHUNK - 2) // 2, main_body, 0)

  g = _NCHUNK - 1
  wait_gather(g, 1)
  scale(1)
  start_store(g, 1)
  wait_store(g - 1, 0)
  wait_store(g, 1)


_gather = pl.kernel(
    _gather_body,
    out_type=jax.ShapeDtypeStruct((_B, _D), jnp.float32),
    mesh=_mesh,
    scratch_types=[
        pltpu.VMEM((_BPW,), jnp.int32),
        pltpu.VMEM((_CHUNK, _D), jnp.float32),
        pltpu.VMEM((_CHUNK, _D), jnp.float32),
        pltpu.SemaphoreType.DMA,
        pltpu.SemaphoreType.DMA,
        pltpu.SemaphoreType.DMA,
        pltpu.SemaphoreType.DMA,
    ],
    compiler_params=pltpu.CompilerParams(use_tc_tiling_on_sc=False),
)


@jax.jit
def kernel(x, table):
  xf = x.reshape(-1).astype(jnp.int32)
  out = _gather(xf, table)
  return out.reshape(x.shape + (_D,))
